# baseline (device time: 232456 ns/iter reference)
import jax
import jax.numpy as jnp
from jax import lax
from jax.experimental import pallas as pl
from jax.experimental.pallas import tpu as pltpu

N_DEV = 4


def kernel(ids, E):
    T = ids.shape[0]
    V_per, D = E.shape

    my = lax.axis_index("i")
    lo = my * V_per
    loc = ids - lo
    mask = (loc >= 0) & (loc < V_per)
    safe = jnp.where(mask, loc, 0)
    partial = jnp.where(
        mask[:, None], E[safe].astype(jnp.bfloat16), jnp.bfloat16(0)
    )

    def body(p_ref, out_ref, comm_ref, send_sems, recv_sems):
        my_pos = lax.axis_index("i")
        left = lax.rem(my_pos + N_DEV - 1, N_DEV)
        right = lax.rem(my_pos + 1, N_DEV)

        barrier_sem = pltpu.get_barrier_semaphore()
        for nbr in (left, right):
            pl.semaphore_signal(
                barrier_sem, inc=1,
                device_id=(nbr,), device_id_type=pl.DeviceIdType.MESH,
            )
        pl.semaphore_wait(barrier_sem, 2)

        out_ref[...] = p_ref[...].astype(jnp.float32)

        for h in range(N_DEV - 1):
            src = p_ref if h == 0 else comm_ref.at[h - 1]
            rdma = pltpu.make_async_remote_copy(
                src_ref=src,
                dst_ref=comm_ref.at[h],
                send_sem=send_sems.at[h],
                recv_sem=recv_sems.at[h],
                device_id=(right,),
                device_id_type=pl.DeviceIdType.MESH,
            )
            rdma.start()
            rdma.wait()
            out_ref[...] += comm_ref[h].astype(jnp.float32)

    return pl.pallas_call(
        body,
        out_shape=jax.ShapeDtypeStruct((T, D), jnp.float32),
        in_specs=[pl.BlockSpec(memory_space=pltpu.VMEM)],
        out_specs=pl.BlockSpec(memory_space=pltpu.VMEM),
        scratch_shapes=[
            pltpu.VMEM((N_DEV - 1, T, D), jnp.bfloat16),
            pltpu.SemaphoreType.DMA((N_DEV - 1,)),
            pltpu.SemaphoreType.DMA((N_DEV - 1,)),
        ],
        compiler_params=pltpu.CompilerParams(collective_id=0),
    )(partial)


# device time: 136257 ns/iter; 1.7060x vs baseline; 1.7060x over previous
import jax
import jax.numpy as jnp
from jax import lax
from jax.experimental import pallas as pl
from jax.experimental.pallas import tpu as pltpu

N_DEV = 4
N_HOP = N_DEV - 1


def kernel(ids, E):
    T = ids.shape[0]
    V_per, D = E.shape
    H = T // 2
    R = H // N_DEV

    my = lax.axis_index("i")
    lo = my * V_per
    loc = ids - lo
    mask = (loc >= 0) & (loc < V_per)
    safe = jnp.where(mask, loc, 0)
    partial = jnp.where(
        mask[:, None], E[safe].astype(jnp.bfloat16), jnp.bfloat16(0)
    )

    def body(p_ref, out_ref, red_ref, rs_buf,
             rs_send, rs_recv, ag_send, ag_recv):
        my_pos = lax.axis_index("i")
        left = lax.rem(my_pos + N_DEV - 1, N_DEV)
        right = lax.rem(my_pos + 1, N_DEV)
        peer = (right, left)

        def cslice(ref, d, idx):
            return ref.at[pl.ds(d * H + idx * R, R)]

        def cidx(off):
            return lax.rem(my_pos + (off % N_DEV), N_DEV)

        barrier_sem = pltpu.get_barrier_semaphore()
        for nbr in (left, right):
            pl.semaphore_signal(
                barrier_sem, inc=1,
                device_id=(nbr,), device_id_type=pl.DeviceIdType.MESH,
            )
        pl.semaphore_wait(barrier_sem, 2)

        red_ref[...] = p_ref[...]

        for h in range(N_HOP):
            rdmas = []
            for d in range(2):
                s_off = (-1 - h) if d == 0 else (1 + h)
                rdma = pltpu.make_async_remote_copy(
                    src_ref=cslice(red_ref, d, cidx(s_off)),
                    dst_ref=rs_buf.at[d, h],
                    send_sem=rs_send.at[d, h],
                    recv_sem=rs_recv.at[d, h],
                    device_id=(peer[d],),
                    device_id_type=pl.DeviceIdType.MESH,
                )
                rdma.start()
                rdmas.append(rdma)
            for d in range(2):
                rdmas[d].wait()
                r_off = (-2 - h) if d == 0 else (2 + h)
                dst = cslice(red_ref, d, cidx(r_off))
                dst[...] = dst[...] + rs_buf[d, h]

        for h in range(N_HOP):
            rdmas = []
            for d in range(2):
                s_off = -h if d == 0 else h
                rdma = pltpu.make_async_remote_copy(
                    src_ref=cslice(red_ref, d, cidx(s_off)),
                    dst_ref=cslice(red_ref, d, cidx(s_off)),
                    send_sem=ag_send.at[d, h],
                    recv_sem=ag_recv.at[d, h],
                    device_id=(peer[d],),
                    device_id_type=pl.DeviceIdType.MESH,
                )
                rdma.start()
                rdmas.append(rdma)
            for rdma in rdmas:
                rdma.wait()

        out_ref[...] = red_ref[...].astype(jnp.float32)

    return pl.pallas_call(
        body,
        out_shape=jax.ShapeDtypeStruct((T, D), jnp.float32),
        in_specs=[pl.BlockSpec(memory_space=pltpu.VMEM)],
        out_specs=pl.BlockSpec(memory_space=pltpu.VMEM),
        scratch_shapes=[
            pltpu.VMEM((T, D), jnp.bfloat16),
            pltpu.VMEM((2, N_HOP, R, D), jnp.bfloat16),
            pltpu.SemaphoreType.DMA((2, N_HOP)),
            pltpu.SemaphoreType.DMA((2, N_HOP)),
            pltpu.SemaphoreType.DMA((2, N_HOP)),
            pltpu.SemaphoreType.DMA((2, N_HOP)),
        ],
        compiler_params=pltpu.CompilerParams(collective_id=0),
    )(partial)


# device time: 114515 ns/iter; 2.0299x vs baseline; 1.1899x over previous
import jax
import jax.numpy as jnp
from jax import lax
from jax.experimental import pallas as pl
from jax.experimental.pallas import tpu as pltpu

N_DEV = 4
N_HOP = N_DEV - 1


def kernel(ids, E):
    T = ids.shape[0]
    V_per, D = E.shape
    H = T // 2
    R = H // N_DEV

    my = lax.axis_index("i")
    loc = ids - my * V_per
    mask = (loc >= 0) & (loc < V_per)
    safe = jnp.where(mask, loc, 0).astype(jnp.int32)
    maskf = mask.astype(jnp.bfloat16)[:, None]

    def body(safe_ref, maskf_ref, e_ref, out_ref, gbuf, red_ref, rs_buf,
             gsem, rs_send, rs_recv, ag_send, ag_recv):
        my_pos = lax.axis_index("i")
        left = lax.rem(my_pos + N_DEV - 1, N_DEV)
        right = lax.rem(my_pos + 1, N_DEV)
        peer = (right, left)

        def cslice(ref, d, idx):
            return ref.at[pl.ds(d * H + idx * R, R)]

        def cidx(off):
            return lax.rem(my_pos + (off % N_DEV), N_DEV)

        barrier_sem = pltpu.get_barrier_semaphore()
        for nbr in (left, right):
            pl.semaphore_signal(
                barrier_sem, inc=1,
                device_id=(nbr,), device_id_type=pl.DeviceIdType.MESH,
            )
        pl.semaphore_wait(barrier_sem, 2)

        def issue(t, _):
            pltpu.make_async_copy(
                e_ref.at[safe_ref[t]], gbuf.at[t], gsem
            ).start()
            return 0

        lax.fori_loop(0, T, issue, 0, unroll=8)

        def drain(t, _):
            pltpu.make_async_copy(e_ref.at[0], gbuf.at[t], gsem).wait()
            return 0

        lax.fori_loop(0, T, drain, 0, unroll=8)

        red_ref[...] = gbuf[...].astype(jnp.bfloat16) * maskf_ref[...]

        for h in range(N_HOP):
            rdmas = []
            for d in range(2):
                s_off = (-1 - h) if d == 0 else (1 + h)
                rdma = pltpu.make_async_remote_copy(
                    src_ref=cslice(red_ref, d, cidx(s_off)),
                    dst_ref=rs_buf.at[d, h],
                    send_sem=rs_send.at[d, h],
                    recv_sem=rs_recv.at[d, h],
                    device_id=(peer[d],),
                    device_id_type=pl.DeviceIdType.MESH,
                )
                rdma.start()
                rdmas.append(rdma)
            for d in range(2):
                rdmas[d].wait()
                r_off = (-2 - h) if d == 0 else (2 + h)
                dst = cslice(red_ref, d, cidx(r_off))
                dst[...] = dst[...] + rs_buf[d, h]

        for h in range(N_HOP):
            rdmas = []
            for d in range(2):
                s_off = -h if d == 0 else h
                rdma = pltpu.make_async_remote_copy(
                    src_ref=cslice(red_ref, d, cidx(s_off)),
                    dst_ref=cslice(red_ref, d, cidx(s_off)),
                    send_sem=ag_send.at[d, h],
                    recv_sem=ag_recv.at[d, h],
                    device_id=(peer[d],),
                    device_id_type=pl.DeviceIdType.MESH,
                )
                rdma.start()
                rdmas.append(rdma)
            for rdma in rdmas:
                rdma.wait()

        out_ref[...] = red_ref[...].astype(jnp.float32)

    return pl.pallas_call(
        body,
        out_shape=jax.ShapeDtypeStruct((T, D), jnp.float32),
        in_specs=[
            pl.BlockSpec(memory_space=pltpu.SMEM),
            pl.BlockSpec(memory_space=pltpu.VMEM),
            pl.BlockSpec(memory_space=pl.ANY),
        ],
        out_specs=pl.BlockSpec(memory_space=pltpu.VMEM),
        scratch_shapes=[
            pltpu.VMEM((T, D), jnp.float32),
            pltpu.VMEM((T, D), jnp.bfloat16),
            pltpu.VMEM((2, N_HOP, R, D), jnp.bfloat16),
            pltpu.SemaphoreType.DMA,
            pltpu.SemaphoreType.DMA((2, N_HOP)),
            pltpu.SemaphoreType.DMA((2, N_HOP)),
            pltpu.SemaphoreType.DMA((2, N_HOP)),
            pltpu.SemaphoreType.DMA((2, N_HOP)),
        ],
        compiler_params=pltpu.CompilerParams(collective_id=0),
    )(safe, maskf, E)


# device time: 95326 ns/iter; 2.4385x vs baseline; 1.2013x over previous
import jax
import jax.numpy as jnp
from jax import lax
from jax.experimental import pallas as pl
from jax.experimental.pallas import tpu as pltpu

N_DEV = 4
N_HOP = N_DEV - 1


def kernel(ids, E):
    T = ids.shape[0]
    V_per, D = E.shape
    H = T // 2
    R = H // N_DEV

    my = lax.axis_index("i")
    loc = ids - my * V_per
    mask = (loc >= 0) & (loc < V_per)
    safe = jnp.where(mask, loc, 0).astype(jnp.int32)
    mask_i = mask.astype(jnp.int32)
    maskf = mask.astype(jnp.bfloat16)[:, None]

    def body(safe_ref, mask_i_ref, maskf_ref, e_ref, out_ref, gbuf, red_ref,
             rs_buf, gsem, rs_send, rs_recv, ag_send, ag_recv):
        my_pos = lax.axis_index("i")
        left = lax.rem(my_pos + N_DEV - 1, N_DEV)
        right = lax.rem(my_pos + 1, N_DEV)
        peer = (right, left)

        def cslice(ref, d, idx):
            return ref.at[pl.ds(d * H + idx * R, R)]

        def cidx(off):
            return lax.rem(my_pos + (off % N_DEV), N_DEV)

        def issue(t, c):
            owned = mask_i_ref[t] > 0

            @pl.when(owned)
            def _():
                pltpu.make_async_copy(
                    e_ref.at[safe_ref[t]], gbuf.at[t], gsem
                ).start()

            return c + jnp.where(owned, 1, 0)

        n_owned = lax.fori_loop(0, T, issue, 0, unroll=16)

        barrier_sem = pltpu.get_barrier_semaphore()
        for nbr in (left, right):
            pl.semaphore_signal(
                barrier_sem, inc=1,
                device_id=(nbr,), device_id_type=pl.DeviceIdType.MESH,
            )
        pl.semaphore_wait(barrier_sem, 2)

        def drain(t, _):
            pltpu.make_async_copy(e_ref.at[0], gbuf.at[0], gsem).wait()
            return 0

        lax.fori_loop(0, n_owned, drain, 0)

        red_ref[...] = jnp.where(
            maskf_ref[...] != 0, gbuf[...].astype(jnp.bfloat16),
            jnp.bfloat16(0),
        )

        for h in range(N_HOP):
            rdmas = []
            for d in range(2):
                s_off = (-1 - h) if d == 0 else (1 + h)
                rdma = pltpu.make_async_remote_copy(
                    src_ref=cslice(red_ref, d, cidx(s_off)),
                    dst_ref=rs_buf.at[d, h],
                    send_sem=rs_send.at[d, h],
                    recv_sem=rs_recv.at[d, h],
                    device_id=(peer[d],),
                    device_id_type=pl.DeviceIdType.MESH,
                )
                rdma.start()
                rdmas.append(rdma)
            for d in range(2):
                rdmas[d].wait()
                r_off = (-2 - h) if d == 0 else (2 + h)
                dst = cslice(red_ref, d, cidx(r_off))
                dst[...] = dst[...] + rs_buf[d, h]

        for h in range(N_HOP):
            rdmas = []
            for d in range(2):
                s_off = -h if d == 0 else h
                rdma = pltpu.make_async_remote_copy(
                    src_ref=cslice(red_ref, d, cidx(s_off)),
                    dst_ref=cslice(red_ref, d, cidx(s_off)),
                    send_sem=ag_send.at[d, h],
                    recv_sem=ag_recv.at[d, h],
                    device_id=(peer[d],),
                    device_id_type=pl.DeviceIdType.MESH,
                )
                rdma.start()
                rdmas.append(rdma)
            for rdma in rdmas:
                rdma.wait()

        out_ref[...] = red_ref[...].astype(jnp.float32)

    return pl.pallas_call(
        body,
        out_shape=jax.ShapeDtypeStruct((T, D), jnp.float32),
        in_specs=[
            pl.BlockSpec(memory_space=pltpu.SMEM),
            pl.BlockSpec(memory_space=pltpu.SMEM),
            pl.BlockSpec(memory_space=pltpu.VMEM),
            pl.BlockSpec(memory_space=pl.ANY),
        ],
        out_specs=pl.BlockSpec(memory_space=pltpu.VMEM),
        scratch_shapes=[
            pltpu.VMEM((T, D), jnp.float32),
            pltpu.VMEM((T, D), jnp.bfloat16),
            pltpu.VMEM((2, N_HOP, R, D), jnp.bfloat16),
            pltpu.SemaphoreType.DMA,
            pltpu.SemaphoreType.DMA((2, N_HOP)),
            pltpu.SemaphoreType.DMA((2, N_HOP)),
            pltpu.SemaphoreType.DMA((2, N_HOP)),
            pltpu.SemaphoreType.DMA((2, N_HOP)),
        ],
        compiler_params=pltpu.CompilerParams(collective_id=0),
    )(safe, mask_i, maskf, E)


# device time: 79280 ns/iter; 2.9321x vs baseline; 1.2024x over previous
import jax
import jax.numpy as jnp
from jax import lax
from jax.experimental import pallas as pl
from jax.experimental.pallas import tpu as pltpu

N_DEV = 4
N_HOP = N_DEV - 1


def kernel(ids, E):
    T = ids.shape[0]
    V_per, D = E.shape
    H = T // 2
    R = H // N_DEV

    my = lax.axis_index("i")
    loc = ids - my * V_per
    mask = (loc >= 0) & (loc < V_per)
    safe = jnp.where(mask, loc, 0).astype(jnp.int32)
    maskf = mask.astype(jnp.bfloat16)[:, None]

    blk = jnp.arange(T, dtype=jnp.int32) // R
    d_half = blk // N_DEV
    c = blk % N_DEV
    rank = jnp.where(
        d_half == 0, jnp.mod(my - 1 - c, N_DEV), jnp.mod(c - my - 1, N_DEV)
    )
    group = 2 * rank + d_half
    key = jnp.where(mask, group, 2 * N_DEV)
    order = jnp.argsort(key, stable=True).astype(jnp.int32)
    dma_src = safe[order]
    dma_dst = order
    counts = jnp.sum(
        jnp.where(mask[None, :], key[None, :] == jnp.arange(8)[:, None], False),
        axis=1,
    )
    cum = jnp.cumsum(counts).astype(jnp.int32)

    def body(dma_src_ref, dma_dst_ref, cum_ref, maskf_ref, e_ref, out_ref,
             gbuf, red_ref, rs_buf, gsem, rs_send, rs_recv, ag_send, ag_recv):
        my_pos = lax.axis_index("i")
        left = lax.rem(my_pos + N_DEV - 1, N_DEV)
        right = lax.rem(my_pos + 1, N_DEV)
        peer = (right, left)

        def cslice(ref, d, idx):
            return ref.at[pl.ds(d * H + idx * R, R)]

        def cidx(off):
            return lax.rem(my_pos + (off % N_DEV), N_DEV)

        n_owned = cum_ref[7]

        def issue(t, _):
            pltpu.make_async_copy(
                e_ref.at[dma_src_ref[t]], gbuf.at[dma_dst_ref[t]], gsem
            ).start()
            return 0

        lax.fori_loop(0, n_owned, issue, 0)

        barrier_sem = pltpu.get_barrier_semaphore()
        for nbr in (left, right):
            pl.semaphore_signal(
                barrier_sem, inc=1,
                device_id=(nbr,), device_id_type=pl.DeviceIdType.MESH,
            )
        pl.semaphore_wait(barrier_sem, 2)

        def drain(t, _):
            pltpu.make_async_copy(e_ref.at[0], gbuf.at[0], gsem).wait()
            return 0

        lax.fori_loop(0, n_owned, drain, 0)

        red_ref[...] = jnp.where(
            maskf_ref[...] != 0, gbuf[...].astype(jnp.bfloat16),
            jnp.bfloat16(0),
        )

        for h in range(N_HOP):
            rdmas = []
            for d in range(2):
                s_off = (-1 - h) if d == 0 else (1 + h)
                rdma = pltpu.make_async_remote_copy(
                    src_ref=cslice(red_ref, d, cidx(s_off)),
                    dst_ref=rs_buf.at[d, h],
                    send_sem=rs_send.at[d, h],
                    recv_sem=rs_recv.at[d, h],
                    device_id=(peer[d],),
                    device_id_type=pl.DeviceIdType.MESH,
                )
                rdma.start()
                rdmas.append(rdma)
            for d in range(2):
                rdmas[d].wait()
                r_off = (-2 - h) if d == 0 else (2 + h)
                dst = cslice(red_ref, d, cidx(r_off))
                dst[...] = dst[...] + rs_buf[d, h]

        for h in range(N_HOP):
            rdmas = []
            for d in range(2):
                s_off = -h if d == 0 else h
                rdma = pltpu.make_async_remote_copy(
                    src_ref=cslice(red_ref, d, cidx(s_off)),
                    dst_ref=cslice(red_ref, d, cidx(s_off)),
                    send_sem=ag_send.at[d, h],
                    recv_sem=ag_recv.at[d, h],
                    device_id=(peer[d],),
                    device_id_type=pl.DeviceIdType.MESH,
                )
                rdma.start()
                rdmas.append(rdma)
            for rdma in rdmas:
                rdma.wait()

        out_ref[...] = red_ref[...].astype(jnp.float32)

    return pl.pallas_call(
        body,
        out_shape=jax.ShapeDtypeStruct((T, D), jnp.float32),
        in_specs=[
            pl.BlockSpec(memory_space=pltpu.SMEM),
            pl.BlockSpec(memory_space=pltpu.SMEM),
            pl.BlockSpec(memory_space=pltpu.SMEM),
            pl.BlockSpec(memory_space=pltpu.VMEM),
            pl.BlockSpec(memory_space=pl.ANY),
        ],
        out_specs=pl.BlockSpec(memory_space=pltpu.VMEM),
        scratch_shapes=[
            pltpu.VMEM((T, D), jnp.float32),
            pltpu.VMEM((T, D), jnp.bfloat16),
            pltpu.VMEM((2, N_HOP, R, D), jnp.bfloat16),
            pltpu.SemaphoreType.DMA,
            pltpu.SemaphoreType.DMA((2, N_HOP)),
            pltpu.SemaphoreType.DMA((2, N_HOP)),
            pltpu.SemaphoreType.DMA((2, N_HOP)),
            pltpu.SemaphoreType.DMA((2, N_HOP)),
        ],
        compiler_params=pltpu.CompilerParams(collective_id=0),
    )(dma_src, dma_dst, cum, maskf, E)
